# Initial kernel scaffold; baseline (speedup 1.0000x reference)
import functools

import jax
import jax.numpy as jnp
from jax import lax
from jax.experimental import pallas as pl
from jax.experimental.pallas import tpu as pltpu
from jax.experimental.pallas import tpu_sc as plsc

N_NODES = 10000
N_EDGES = 320000
D_FEAT = 128
D_EDGE = 16

NC = 2
NS = 16
NW = NC * NS

IDXW = 128
ROWS = N_EDGES // IDXW
ROWS_PER_TILE = ROWS // NW
ROWS_MAIN = ROWS_PER_TILE * NW
ROWS_REM = ROWS - ROWS_MAIN
CHUNK_ROWS = 39
NCHUNKS = ROWS_PER_TILE // CHUNK_ROWS
NODES_PER_TILE = N_NODES // NS


def _sc_segment_sums(edges, senders2d, receivers2d):
    mesh = plsc.VectorSubcoreMesh(core_axis_name="c", subcore_axis_name="s")

    @functools.partial(
        pl.kernel,
        out_type=jax.ShapeDtypeStruct((NC, 2, N_NODES, D_EDGE), jnp.float32),
        mesh=mesh,
        scratch_types=[
            pltpu.VMEM((CHUNK_ROWS * IDXW, D_EDGE), jnp.float32),
            pltpu.VMEM((CHUNK_ROWS, IDXW), jnp.int32),
            pltpu.VMEM((CHUNK_ROWS, IDXW), jnp.int32),
            pltpu.VMEM((NODES_PER_TILE, D_EDGE), jnp.float32),
            pltpu.VMEM_SHARED((N_NODES, D_EDGE), jnp.float32),
            pltpu.VMEM_SHARED((N_NODES, D_EDGE), jnp.float32),
        ],
    )
    def seg_kernel(edges_hbm, s_hbm, r_hbm, out_hbm,
                   ebuf, sbuf, rbuf, tbuf, sent_acc, recv_acc):
        cid = lax.axis_index("c")
        sid = lax.axis_index("s")
        wid = cid * NS + sid

        @pl.loop(0, NODES_PER_TILE)
        def _(i):
            tbuf[i, :] = jnp.zeros((D_EDGE,), jnp.float32)

        base = sid * NODES_PER_TILE
        pltpu.sync_copy(tbuf, sent_acc.at[pl.ds(base, NODES_PER_TILE)])
        pltpu.sync_copy(tbuf, recv_acc.at[pl.ds(base, NODES_PER_TILE)])
        plsc.subcore_barrier()

        def process(row0, nrows):
            e0 = row0 * IDXW
            pltpu.sync_copy(edges_hbm.at[pl.ds(e0, nrows * IDXW)],
                            ebuf.at[pl.ds(0, nrows * IDXW)])
            pltpu.sync_copy(s_hbm.at[pl.ds(row0, nrows)],
                            sbuf.at[pl.ds(0, nrows)])
            pltpu.sync_copy(r_hbm.at[pl.ds(row0, nrows)],
                            rbuf.at[pl.ds(0, nrows)])

            @pl.loop(0, nrows)
            def _(j):
                e = ebuf.at[pl.ds(j * IDXW, IDXW)]
                pltpu.sync_copy(e, sent_acc.at[sbuf.at[j]], add=True)
                pltpu.sync_copy(e, recv_acc.at[rbuf.at[j]], add=True)

        @pl.loop(0, NCHUNKS)
        def _(ci):
            process(wid * ROWS_PER_TILE + ci * CHUNK_ROWS, CHUNK_ROWS)

        @pl.when(wid < ROWS_REM)
        def _():
            process(ROWS_MAIN + wid, 1)

        plsc.subcore_barrier()

        pltpu.sync_copy(sent_acc.at[pl.ds(base, NODES_PER_TILE)], tbuf)
        pltpu.sync_copy(tbuf, out_hbm.at[cid, 0, pl.ds(base, NODES_PER_TILE)])
        pltpu.sync_copy(recv_acc.at[pl.ds(base, NODES_PER_TILE)], tbuf)
        pltpu.sync_copy(tbuf, out_hbm.at[cid, 1, pl.ds(base, NODES_PER_TILE)])

    return seg_kernel(edges, senders2d, receivers2d)


BLK = 1000


def _mlp_kernel(nodes_ref, p_ref, w1n_ref, w1s_ref, w1r_ref, b1_ref,
                w2_ref, b2_ref, out_ref):
    p = p_ref[...]
    sent = p[0, 0] + p[1, 0]
    recv = p[0, 1] + p[1, 1]
    h = (jnp.dot(nodes_ref[...], w1n_ref[...], preferred_element_type=jnp.float32)
         + jnp.dot(sent, w1s_ref[...], preferred_element_type=jnp.float32)
         + jnp.dot(recv, w1r_ref[...], preferred_element_type=jnp.float32)
         + b1_ref[...])
    h = jnp.maximum(h, 0.0)
    out_ref[...] = (jnp.dot(h, w2_ref[...], preferred_element_type=jnp.float32)
                    + b2_ref[...])


def _tc_mlp(nodes, partials, W1, b1, W2, b2):
    w1n = W1[:D_FEAT]
    w1s = W1[D_FEAT:D_FEAT + D_EDGE]
    w1r = W1[D_FEAT + D_EDGE:]
    b1r = b1.reshape(1, -1)
    b2r = b2.reshape(1, -1)
    grid = N_NODES // BLK
    return pl.pallas_call(
        _mlp_kernel,
        grid=(grid,),
        in_specs=[
            pl.BlockSpec((BLK, D_FEAT), lambda i: (i, 0)),
            pl.BlockSpec((NC, 2, BLK, D_EDGE), lambda i: (0, 0, i, 0)),
            pl.BlockSpec((D_FEAT, D_FEAT), lambda i: (0, 0)),
            pl.BlockSpec((D_EDGE, D_FEAT), lambda i: (0, 0)),
            pl.BlockSpec((D_EDGE, D_FEAT), lambda i: (0, 0)),
            pl.BlockSpec((1, D_FEAT), lambda i: (0, 0)),
            pl.BlockSpec((D_FEAT, D_FEAT), lambda i: (0, 0)),
            pl.BlockSpec((1, D_FEAT), lambda i: (0, 0)),
        ],
        out_specs=pl.BlockSpec((BLK, D_FEAT), lambda i: (i, 0)),
        out_shape=jax.ShapeDtypeStruct((N_NODES, D_FEAT), jnp.float32),
    )(nodes, partials, w1n, w1s, w1r, b1r, W2, b2r)


def kernel(nodes, edges, senders, receivers, W1, b1, W2, b2):
    senders2d = senders.astype(jnp.int32).reshape(ROWS, IDXW)
    receivers2d = receivers.astype(jnp.int32).reshape(ROWS, IDXW)
    partials = _sc_segment_sums(edges, senders2d, receivers2d)
    return _tc_mlp(nodes, partials, W1, b1, W2, b2)


# speedup vs baseline: 7.7145x; 7.7145x over previous
import functools

import jax
import jax.numpy as jnp
from jax import lax
from jax.experimental import pallas as pl
from jax.experimental.pallas import tpu as pltpu
from jax.experimental.pallas import tpu_sc as plsc

N_NODES = 10000
N_EDGES = 320000
D_FEAT = 128
D_EDGE = 16

NC = 2
NS = 16
NW = NC * NS

IDXW = 128
ROWS = N_EDGES // IDXW
ROWS_PER_TILE = ROWS // NW
ROWS_MAIN = ROWS_PER_TILE * NW
ROWS_REM = ROWS - ROWS_MAIN
CHUNK_ROWS = 39
NCHUNKS = ROWS_PER_TILE // CHUNK_ROWS

ACC_ROWS = 10240
STRIPE = ACC_ROWS // NS
LAST_STRIPE = N_NODES - (NS - 1) * STRIPE


def _sc_segment_sums(edges, senders3d, receivers3d):
    mesh = plsc.VectorSubcoreMesh(core_axis_name="c", subcore_axis_name="s")

    @functools.partial(
        pl.kernel,
        out_type=jax.ShapeDtypeStruct((NC, 2, N_NODES, D_EDGE), jnp.float32),
        mesh=mesh,
        compiler_params=pltpu.CompilerParams(use_tc_tiling_on_sc=False),
        scratch_types=[
            pltpu.VMEM((CHUNK_ROWS * IDXW, D_EDGE), jnp.float32),
            pltpu.VMEM((CHUNK_ROWS, 1, IDXW), jnp.int32),
            pltpu.VMEM((CHUNK_ROWS, 1, IDXW), jnp.int32),
            pltpu.VMEM((STRIPE, D_EDGE), jnp.float32),
            pltpu.VMEM_SHARED((ACC_ROWS, D_EDGE), jnp.float32),
            pltpu.VMEM_SHARED((ACC_ROWS, D_EDGE), jnp.float32),
        ],
    )
    def seg_kernel(edges_hbm, s_hbm, r_hbm, out_hbm,
                   ebuf, sbuf, rbuf, tbuf, sent_acc, recv_acc):
        cid = lax.axis_index("c")
        sid = lax.axis_index("s")
        wid = cid * NS + sid

        @pl.loop(0, STRIPE)
        def _(i):
            tbuf[i, :] = jnp.zeros((D_EDGE,), jnp.float32)

        base = sid * STRIPE
        pltpu.sync_copy(tbuf, sent_acc.at[pl.ds(base, STRIPE)])
        pltpu.sync_copy(tbuf, recv_acc.at[pl.ds(base, STRIPE)])
        plsc.subcore_barrier()

        def process(row0, nrows):
            e0 = row0 * IDXW
            pltpu.sync_copy(edges_hbm.at[pl.ds(e0, nrows * IDXW)],
                            ebuf.at[pl.ds(0, nrows * IDXW)])
            pltpu.sync_copy(s_hbm.at[pl.ds(row0, nrows)],
                            sbuf.at[pl.ds(0, nrows)])
            pltpu.sync_copy(r_hbm.at[pl.ds(row0, nrows)],
                            rbuf.at[pl.ds(0, nrows)])

            @pl.loop(0, nrows)
            def _(j):
                e = ebuf.at[pl.ds(j * IDXW, IDXW)]
                pltpu.sync_copy(e, sent_acc.at[sbuf.at[j, 0]], add=True)
                pltpu.sync_copy(e, recv_acc.at[rbuf.at[j, 0]], add=True)

        @pl.loop(0, NCHUNKS)
        def _(ci):
            process(wid * ROWS_PER_TILE + ci * CHUNK_ROWS, CHUNK_ROWS)

        @pl.when(wid < ROWS_REM)
        def _():
            process(ROWS_MAIN + wid, 1)

        plsc.subcore_barrier()

        def copy_out(acc, agg, rows):
            pltpu.sync_copy(acc.at[pl.ds(base, rows)], tbuf.at[pl.ds(0, rows)])
            pltpu.sync_copy(tbuf.at[pl.ds(0, rows)],
                            out_hbm.at[cid, agg, pl.ds(base, rows)])

        @pl.when(sid < NS - 1)
        def _():
            copy_out(sent_acc, 0, STRIPE)
            copy_out(recv_acc, 1, STRIPE)

        @pl.when(sid == NS - 1)
        def _():
            copy_out(sent_acc, 0, LAST_STRIPE)
            copy_out(recv_acc, 1, LAST_STRIPE)

    return seg_kernel(edges, senders3d, receivers3d)


BLK = 1000


def _mlp_kernel(nodes_ref, p_ref, w1n_ref, w1s_ref, w1r_ref, b1_ref,
                w2_ref, b2_ref, out_ref):
    p = p_ref[...]
    sent = p[0, 0] + p[1, 0]
    recv = p[0, 1] + p[1, 1]
    h = (jnp.dot(nodes_ref[...], w1n_ref[...], preferred_element_type=jnp.float32)
         + jnp.dot(sent, w1s_ref[...], preferred_element_type=jnp.float32)
         + jnp.dot(recv, w1r_ref[...], preferred_element_type=jnp.float32)
         + b1_ref[...])
    h = jnp.maximum(h, 0.0)
    out_ref[...] = (jnp.dot(h, w2_ref[...], preferred_element_type=jnp.float32)
                    + b2_ref[...])


def _tc_mlp(nodes, partials, W1, b1, W2, b2):
    w1n = W1[:D_FEAT]
    w1s = W1[D_FEAT:D_FEAT + D_EDGE]
    w1r = W1[D_FEAT + D_EDGE:]
    b1r = b1.reshape(1, -1)
    b2r = b2.reshape(1, -1)
    grid = N_NODES // BLK
    return pl.pallas_call(
        _mlp_kernel,
        grid=(grid,),
        in_specs=[
            pl.BlockSpec((BLK, D_FEAT), lambda i: (i, 0)),
            pl.BlockSpec((NC, 2, BLK, D_EDGE), lambda i: (0, 0, i, 0)),
            pl.BlockSpec((D_FEAT, D_FEAT), lambda i: (0, 0)),
            pl.BlockSpec((D_EDGE, D_FEAT), lambda i: (0, 0)),
            pl.BlockSpec((D_EDGE, D_FEAT), lambda i: (0, 0)),
            pl.BlockSpec((1, D_FEAT), lambda i: (0, 0)),
            pl.BlockSpec((D_FEAT, D_FEAT), lambda i: (0, 0)),
            pl.BlockSpec((1, D_FEAT), lambda i: (0, 0)),
        ],
        out_specs=pl.BlockSpec((BLK, D_FEAT), lambda i: (i, 0)),
        out_shape=jax.ShapeDtypeStruct((N_NODES, D_FEAT), jnp.float32),
    )(nodes, partials, w1n, w1s, w1r, b1r, W2, b2r)


def kernel(nodes, edges, senders, receivers, W1, b1, W2, b2):
    senders3d = senders.astype(jnp.int32).reshape(ROWS, 1, IDXW)
    receivers3d = receivers.astype(jnp.int32).reshape(ROWS, 1, IDXW)
    partials = _sc_segment_sums(edges, senders3d, receivers3d)
    return _tc_mlp(nodes, partials, W1, b1, W2, b2)


# speedup vs baseline: 16.3530x; 2.1198x over previous
import functools

import jax
import jax.numpy as jnp
from jax import lax
from jax.experimental import pallas as pl
from jax.experimental.pallas import tpu as pltpu
from jax.experimental.pallas import tpu_sc as plsc

N_NODES = 10000
N_EDGES = 320000
D_FEAT = 128
D_EDGE = 16

NC = 2
NS = 16
NW = NC * NS

IDXW = 128
GROUPS = N_EDGES // IDXW
GROUPS_PER_TILE = GROUPS // NW
GROUPS_MAIN = GROUPS_PER_TILE * NW
GROUPS_REM = GROUPS - GROUPS_MAIN
CHUNK = 6
NCHUNKS = GROUPS_PER_TILE // CHUNK
XROWS = CHUNK * 8
HALF_OFF = N_EDGES // D_EDGE

ACC_ROWS = 10240
STRIPE = ACC_ROWS // NS


def _sc_segment_sums(edges_nat, senders, receivers):
    mesh = plsc.VectorSubcoreMesh(core_axis_name="c", subcore_axis_name="s")

    out_t = jax.ShapeDtypeStruct((ACC_ROWS, D_EDGE), jnp.float32)
    XLEN = 2 * CHUNK * 8 * 128
    ILEN = CHUNK * IDXW
    SCAT_BYTES = IDXW * D_EDGE * 4

    @functools.partial(
        pl.kernel,
        out_type=[out_t, out_t, out_t, out_t],
        mesh=mesh,
        compiler_params=pltpu.CompilerParams(use_tc_tiling_on_sc=False,
                                             needs_layout_passes=False),
        scratch_types=[
            pltpu.VMEM((XLEN,), jnp.float32),
            pltpu.VMEM((XLEN,), jnp.float32),
            pltpu.VMEM((ILEN, D_EDGE), jnp.float32),
            pltpu.VMEM((ILEN, D_EDGE), jnp.float32),
            pltpu.VMEM((ILEN,), jnp.int32),
            pltpu.VMEM((ILEN,), jnp.int32),
            pltpu.VMEM((ILEN,), jnp.int32),
            pltpu.VMEM((ILEN,), jnp.int32),
            pltpu.VMEM((ILEN,), jnp.int32),
            pltpu.VMEM((ILEN,), jnp.int32),
            pltpu.VMEM((STRIPE, D_EDGE), jnp.float32),
            pltpu.VMEM_SHARED((ACC_ROWS, D_EDGE), jnp.float32),
            pltpu.VMEM_SHARED((ACC_ROWS, D_EDGE), jnp.float32),
            pltpu.SemaphoreType.DMA,
            pltpu.SemaphoreType.DMA,
            pltpu.SemaphoreType.DMA,
        ],
    )
    def seg_kernel(edges_hbm, s_hbm, r_hbm, out_s0, out_r0, out_s1, out_r1,
                   xbuf0, xbuf1, ebuf0, ebuf1, sb0, sb1, sb2, rb0, rb1, rb2,
                   tbuf, sent_acc, recv_acc, sem_in, sem_sc0, sem_sc1):
        cid = lax.axis_index("c")
        sid = lax.axis_index("s")
        wid = cid * NS + sid

        xbuf = [xbuf0, xbuf1]
        ebuf = [ebuf0, ebuf1]
        sbuf = [sb0, sb1, sb2]
        rbuf = [rb0, rb1, rb2]
        sem_sc = [sem_sc0, sem_sc1]

        @pl.loop(0, STRIPE)
        def _(i):
            tbuf[i, :] = jnp.zeros((D_EDGE,), jnp.float32)

        base = sid * STRIPE
        pltpu.sync_copy(tbuf, sent_acc.at[pl.ds(base, STRIPE)])
        pltpu.sync_copy(tbuf, recv_acc.at[pl.ds(base, STRIPE)])
        plsc.subcore_barrier()

        iota = lax.iota(jnp.int32, 16)

        def issue_inputs(ci):
            g0 = wid * GROUPS_PER_TILE + ci * CHUNK
            e0 = g0 * IDXW
            xb = xbuf[ci % 2]
            return [
                pltpu.async_copy(edges_hbm.at[pl.ds(1024 * g0, 1024 * CHUNK)],
                                 xb.at[pl.ds(0, 1024 * CHUNK)], sem_in),
                pltpu.async_copy(
                    edges_hbm.at[pl.ds(128 * HALF_OFF + 1024 * g0,
                                       1024 * CHUNK)],
                    xb.at[pl.ds(CHUNK * 1024, 1024 * CHUNK)], sem_in),
                pltpu.async_copy(s_hbm.at[pl.ds(e0, ILEN)], sbuf[ci % 3],
                                 sem_in),
                pltpu.async_copy(r_hbm.at[pl.ds(e0, ILEN)], rbuf[ci % 3],
                                 sem_in),
            ]

        def drain_scatters(parity):
            for _ in range(2):
                pltpu.make_async_copy(edges_hbm.at[pl.ds(0, XLEN)],
                                      xbuf[0], sem_sc[parity]).wait()
        assert 2 * CHUNK * SCAT_BYTES == 2 * XLEN * 4

        in_handles = issue_inputs(0)
        for ci in range(NCHUNKS):
            b = ci % 2
            for h in in_handles:
                h.wait()
            if ci >= 2:
                drain_scatters(b)
            if ci + 1 < NCHUNKS:
                in_handles = issue_inputs(ci + 1)
            xb, eb = xbuf[b], ebuf[b]
            sb, rb = sbuf[ci % 3], rbuf[ci % 3]

            @pl.loop(0, CHUNK)
            def _(j):
                for d in range(D_EDGE):
                    xoff = (d >> 3) * (CHUNK * 1024) + (d & 7) * 128
                    cvec = jnp.full((16,), d, jnp.int32)
                    for c0 in range(IDXW // 16):
                        v = xb[pl.ds(1024 * j + xoff + 16 * c0, 16)]
                        dst = eb.at[pl.ds(IDXW * j + 16 * c0, 16)]
                        plsc.store_scatter(dst, [iota, cvec], v)
                e = eb.at[pl.ds(j * IDXW, IDXW)]
                pltpu.async_copy(e, sent_acc.at[sb.at[pl.ds(j * IDXW, IDXW)]],
                                 sem_sc[b], add=True)
                pltpu.async_copy(e, recv_acc.at[rb.at[pl.ds(j * IDXW, IDXW)]],
                                 sem_sc[b], add=True)

        drain_scatters(NCHUNKS % 2)
        drain_scatters((NCHUNKS - 1) % 2)

        @pl.when(wid < GROUPS_REM)
        def _():
            g0 = GROUPS_MAIN + wid
            e0 = g0 * IDXW
            pltpu.sync_copy(edges_hbm.at[pl.ds(1024 * g0, 1024)],
                            xbuf0.at[pl.ds(0, 1024)])
            pltpu.sync_copy(
                edges_hbm.at[pl.ds(128 * HALF_OFF + 1024 * g0, 1024)],
                xbuf0.at[pl.ds(CHUNK * 1024, 1024)])
            pltpu.sync_copy(s_hbm.at[pl.ds(e0, IDXW)],
                            sb0.at[pl.ds(0, IDXW)])
            pltpu.sync_copy(r_hbm.at[pl.ds(e0, IDXW)],
                            rb0.at[pl.ds(0, IDXW)])
            for d in range(D_EDGE):
                xoff = (d >> 3) * (CHUNK * 1024) + (d & 7) * 128
                cvec = jnp.full((16,), d, jnp.int32)
                for c0 in range(IDXW // 16):
                    v = xbuf0[pl.ds(xoff + 16 * c0, 16)]
                    plsc.store_scatter(ebuf0.at[pl.ds(16 * c0, 16)],
                                       [iota, cvec], v)
            e = ebuf0.at[pl.ds(0, IDXW)]
            pltpu.sync_copy(e, sent_acc.at[sb0.at[pl.ds(0, IDXW)]], add=True)
            pltpu.sync_copy(e, recv_acc.at[rb0.at[pl.ds(0, IDXW)]], add=True)

        plsc.subcore_barrier()

        def copy_out(acc, out_ref):
            pltpu.sync_copy(acc.at[pl.ds(base, STRIPE)], tbuf)
            pltpu.sync_copy(tbuf, out_ref.at[pl.ds(base, STRIPE)])

        @pl.when(cid == 0)
        def _():
            copy_out(sent_acc, out_s0)
            copy_out(recv_acc, out_r0)

        @pl.when(cid == 1)
        def _():
            copy_out(sent_acc, out_s1)
            copy_out(recv_acc, out_r1)

    return seg_kernel(edges_nat, senders, receivers)


BLK = 1024
PACK = D_FEAT // D_EDGE
PBLK = BLK // PACK


def _mlp_kernel(nodes_ref, s0_ref, s1_ref, r0_ref, r1_ref,
                w1n_ref, wbs_ref, wbr_ref, b1_ref, w2_ref, b2_ref, out_ref):
    sent = s0_ref[...] + s1_ref[...]
    recv = r0_ref[...] + r1_ref[...]
    hs = jnp.dot(sent, wbs_ref[...],
                 preferred_element_type=jnp.float32).reshape(BLK, D_FEAT)
    hr = jnp.dot(recv, wbr_ref[...],
                 preferred_element_type=jnp.float32).reshape(BLK, D_FEAT)
    h = (jnp.dot(nodes_ref[...], w1n_ref[...],
                 preferred_element_type=jnp.float32) + hs + hr + b1_ref[...])
    h = jnp.maximum(h, 0.0)
    out_ref[...] = (jnp.dot(h, w2_ref[...], preferred_element_type=jnp.float32)
                    + b2_ref[...])


def _tc_mlp(nodes, s0, r0, s1, r1, W1, b1, W2, b2):
    w1n = W1[:D_FEAT]
    w1s = W1[D_FEAT:D_FEAT + D_EDGE]
    w1r = W1[D_FEAT + D_EDGE:]
    eye8 = jnp.eye(PACK, dtype=jnp.float32)
    wbs = jnp.einsum("ab,dj->adbj", eye8, w1s).reshape(D_FEAT, PACK * D_FEAT)
    wbr = jnp.einsum("ab,dj->adbj", eye8, w1r).reshape(D_FEAT, PACK * D_FEAT)
    b1r = b1.reshape(1, -1)
    b2r = b2.reshape(1, -1)
    grid = (N_NODES + BLK - 1) // BLK
    pk_spec = pl.BlockSpec((PBLK, D_FEAT), lambda i: (i, 0))
    full = lambda shape: pl.BlockSpec(shape, lambda i: tuple(0 for _ in shape))
    return pl.pallas_call(
        _mlp_kernel,
        grid=(grid,),
        in_specs=[
            pl.BlockSpec((BLK, D_FEAT), lambda i: (i, 0)),
            pk_spec, pk_spec, pk_spec, pk_spec,
            full((D_FEAT, D_FEAT)),
            full((D_FEAT, PACK * D_FEAT)),
            full((D_FEAT, PACK * D_FEAT)),
            full((1, D_FEAT)),
            full((D_FEAT, D_FEAT)),
            full((1, D_FEAT)),
        ],
        out_specs=pl.BlockSpec((BLK, D_FEAT), lambda i: (i, 0)),
        out_shape=jax.ShapeDtypeStruct((N_NODES, D_FEAT), jnp.float32),
    )(nodes, s0, s1, r0, r1, w1n, wbs, wbr, b1r, W2, b2r)


def kernel(nodes, edges, senders, receivers, W1, b1, W2, b2):
    edges_nat = (edges.T.reshape(2, 8, GROUPS, 128)
                 .transpose(0, 2, 1, 3).reshape(-1))
    s0, r0, s1, r1 = _sc_segment_sums(edges_nat, senders.astype(jnp.int32),
                                      receivers.astype(jnp.int32))
    pk = ACC_ROWS * D_EDGE // D_FEAT
    return _tc_mlp(nodes, s0.reshape(pk, D_FEAT), r0.reshape(pk, D_FEAT),
                   s1.reshape(pk, D_FEAT), r1.reshape(pk, D_FEAT),
                   W1, b1, W2, b2)


# speedup vs baseline: 20.2971x; 1.2412x over previous
import functools

import jax
import jax.numpy as jnp
from jax import lax
from jax.experimental import pallas as pl
from jax.experimental.pallas import tpu as pltpu
from jax.experimental.pallas import tpu_sc as plsc

N_NODES = 10000
N_EDGES = 320000
D_FEAT = 128
D_EDGE = 16

NC = 2
NS = 16
NW = NC * NS

IDXW = 128
GROUPS = N_EDGES // IDXW
GROUPS_PER_TILE = GROUPS // NW
GROUPS_MAIN = GROUPS_PER_TILE * NW
GROUPS_REM = GROUPS - GROUPS_MAIN
CHUNK = 6
NCHUNKS = GROUPS_PER_TILE // CHUNK
XROWS = CHUNK * 8
HALF_OFF = N_EDGES // D_EDGE

ACC_ROWS = 10240
STRIPE = ACC_ROWS // NS


def _sc_segment_sums(edges_nat, senders, receivers):
    mesh = plsc.VectorSubcoreMesh(core_axis_name="c", subcore_axis_name="s")

    out_t = jax.ShapeDtypeStruct((ACC_ROWS, D_EDGE), jnp.float32)
    XLEN = 2 * CHUNK * 8 * 128
    ILEN = CHUNK * IDXW
    SCAT_BYTES = IDXW * D_EDGE * 4

    @functools.partial(
        pl.kernel,
        out_type=[out_t, out_t, out_t, out_t],
        mesh=mesh,
        compiler_params=pltpu.CompilerParams(use_tc_tiling_on_sc=False,
                                             needs_layout_passes=False),
        scratch_types=[
            pltpu.VMEM((XLEN,), jnp.float32),
            pltpu.VMEM((XLEN,), jnp.float32),
            pltpu.VMEM((ILEN, D_EDGE), jnp.float32),
            pltpu.VMEM((ILEN, D_EDGE), jnp.float32),
            pltpu.VMEM((ILEN,), jnp.int32),
            pltpu.VMEM((ILEN,), jnp.int32),
            pltpu.VMEM((ILEN,), jnp.int32),
            pltpu.VMEM((ILEN,), jnp.int32),
            pltpu.VMEM((ILEN,), jnp.int32),
            pltpu.VMEM((ILEN,), jnp.int32),
            pltpu.VMEM((STRIPE, D_EDGE), jnp.float32),
            pltpu.VMEM_SHARED((ACC_ROWS, D_EDGE), jnp.float32),
            pltpu.VMEM_SHARED((ACC_ROWS, D_EDGE), jnp.float32),
            pltpu.SemaphoreType.DMA,
            pltpu.SemaphoreType.DMA,
            pltpu.SemaphoreType.DMA,
        ],
    )
    def seg_kernel(edges_hbm, s_hbm, r_hbm, out_s0, out_r0, out_s1, out_r1,
                   xbuf0, xbuf1, ebuf0, ebuf1, sb0, sb1, sb2, rb0, rb1, rb2,
                   tbuf, sent_acc, recv_acc, sem_in, sem_sc0, sem_sc1):
        cid = lax.axis_index("c")
        sid = lax.axis_index("s")
        wid = cid * NS + sid

        xbuf = [xbuf0, xbuf1]
        ebuf = [ebuf0, ebuf1]
        sbuf = [sb0, sb1, sb2]
        rbuf = [rb0, rb1, rb2]
        sem_sc = [sem_sc0, sem_sc1]

        @pl.loop(0, STRIPE)
        def _(i):
            tbuf[i, :] = jnp.zeros((D_EDGE,), jnp.float32)

        base = sid * STRIPE
        pltpu.sync_copy(tbuf, sent_acc.at[pl.ds(base, STRIPE)])
        pltpu.sync_copy(tbuf, recv_acc.at[pl.ds(base, STRIPE)])
        plsc.subcore_barrier()

        iota = lax.iota(jnp.int32, 16)

        def issue_inputs(ci):
            g0 = wid * GROUPS_PER_TILE + ci * CHUNK
            e0 = g0 * IDXW
            xb = xbuf[ci % 2]
            return [
                pltpu.async_copy(edges_hbm.at[pl.ds(1024 * g0, 1024 * CHUNK)],
                                 xb.at[pl.ds(0, 1024 * CHUNK)], sem_in),
                pltpu.async_copy(
                    edges_hbm.at[pl.ds(128 * HALF_OFF + 1024 * g0,
                                       1024 * CHUNK)],
                    xb.at[pl.ds(CHUNK * 1024, 1024 * CHUNK)], sem_in),
                pltpu.async_copy(s_hbm.at[pl.ds(e0, ILEN)], sbuf[ci % 3],
                                 sem_in),
                pltpu.async_copy(r_hbm.at[pl.ds(e0, ILEN)], rbuf[ci % 3],
                                 sem_in),
            ]

        def drain_scatters(parity):
            for _ in range(2):
                pltpu.make_async_copy(edges_hbm.at[pl.ds(0, XLEN)],
                                      xbuf[0], sem_sc[parity]).wait()
        assert 2 * CHUNK * SCAT_BYTES == 2 * XLEN * 4

        in_handles = issue_inputs(0)
        for ci in range(NCHUNKS):
            b = ci % 2
            for h in in_handles:
                h.wait()
            if ci >= 2:
                drain_scatters(b)
            if ci + 1 < NCHUNKS:
                in_handles = issue_inputs(ci + 1)
            xb, eb = xbuf[b], ebuf[b]
            sb, rb = sbuf[ci % 3], rbuf[ci % 3]

            @pl.loop(0, CHUNK)
            def _(j):
                @plsc.parallel_loop(0, IDXW // 16)
                def _(c0):
                    for d in range(D_EDGE):
                        xoff = (d >> 3) * (CHUNK * 1024) + (d & 7) * 128
                        cvec = jnp.full((16,), d, jnp.int32)
                        v = xb[pl.ds(1024 * j + xoff + 16 * c0, 16)]
                        dst = eb.at[pl.ds(IDXW * j + 16 * c0, 16)]
                        plsc.store_scatter(dst, [iota, cvec], v)

            @pl.loop(0, CHUNK)
            def _(j):
                e = eb.at[pl.ds(j * IDXW, IDXW)]
                pltpu.async_copy(e, sent_acc.at[sb.at[pl.ds(j * IDXW, IDXW)]],
                                 sem_sc[b], add=True)
                pltpu.async_copy(e, recv_acc.at[rb.at[pl.ds(j * IDXW, IDXW)]],
                                 sem_sc[b], add=True)

        drain_scatters(NCHUNKS % 2)
        drain_scatters((NCHUNKS - 1) % 2)

        @pl.when(wid < GROUPS_REM)
        def _():
            g0 = GROUPS_MAIN + wid
            e0 = g0 * IDXW
            pltpu.sync_copy(edges_hbm.at[pl.ds(1024 * g0, 1024)],
                            xbuf0.at[pl.ds(0, 1024)])
            pltpu.sync_copy(
                edges_hbm.at[pl.ds(128 * HALF_OFF + 1024 * g0, 1024)],
                xbuf0.at[pl.ds(CHUNK * 1024, 1024)])
            pltpu.sync_copy(s_hbm.at[pl.ds(e0, IDXW)],
                            sb0.at[pl.ds(0, IDXW)])
            pltpu.sync_copy(r_hbm.at[pl.ds(e0, IDXW)],
                            rb0.at[pl.ds(0, IDXW)])
            for d in range(D_EDGE):
                xoff = (d >> 3) * (CHUNK * 1024) + (d & 7) * 128
                cvec = jnp.full((16,), d, jnp.int32)
                for c0 in range(IDXW // 16):
                    v = xbuf0[pl.ds(xoff + 16 * c0, 16)]
                    plsc.store_scatter(ebuf0.at[pl.ds(16 * c0, 16)],
                                       [iota, cvec], v)
            e = ebuf0.at[pl.ds(0, IDXW)]
            pltpu.sync_copy(e, sent_acc.at[sb0.at[pl.ds(0, IDXW)]], add=True)
            pltpu.sync_copy(e, recv_acc.at[rb0.at[pl.ds(0, IDXW)]], add=True)

        plsc.subcore_barrier()

        def copy_out(acc, out_ref):
            pltpu.sync_copy(acc.at[pl.ds(base, STRIPE)], tbuf)
            pltpu.sync_copy(tbuf, out_ref.at[pl.ds(base, STRIPE)])

        @pl.when(cid == 0)
        def _():
            copy_out(sent_acc, out_s0)
            copy_out(recv_acc, out_r0)

        @pl.when(cid == 1)
        def _():
            copy_out(sent_acc, out_s1)
            copy_out(recv_acc, out_r1)

    return seg_kernel(edges_nat, senders, receivers)


BLK = 1024
PACK = D_FEAT // D_EDGE
PBLK = BLK // PACK


def _mlp_kernel(nodes_ref, s0_ref, s1_ref, r0_ref, r1_ref,
                w1n_ref, wbs_ref, wbr_ref, b1_ref, w2_ref, b2_ref, out_ref):
    sent = s0_ref[...] + s1_ref[...]
    recv = r0_ref[...] + r1_ref[...]
    hs = jnp.dot(sent, wbs_ref[...],
                 preferred_element_type=jnp.float32).reshape(BLK, D_FEAT)
    hr = jnp.dot(recv, wbr_ref[...],
                 preferred_element_type=jnp.float32).reshape(BLK, D_FEAT)
    h = (jnp.dot(nodes_ref[...], w1n_ref[...],
                 preferred_element_type=jnp.float32) + hs + hr + b1_ref[...])
    h = jnp.maximum(h, 0.0)
    out_ref[...] = (jnp.dot(h, w2_ref[...], preferred_element_type=jnp.float32)
                    + b2_ref[...])


def _tc_mlp(nodes, s0, r0, s1, r1, W1, b1, W2, b2):
    w1n = W1[:D_FEAT]
    w1s = W1[D_FEAT:D_FEAT + D_EDGE]
    w1r = W1[D_FEAT + D_EDGE:]
    eye8 = jnp.eye(PACK, dtype=jnp.float32)
    wbs = jnp.einsum("ab,dj->adbj", eye8, w1s).reshape(D_FEAT, PACK * D_FEAT)
    wbr = jnp.einsum("ab,dj->adbj", eye8, w1r).reshape(D_FEAT, PACK * D_FEAT)
    b1r = b1.reshape(1, -1)
    b2r = b2.reshape(1, -1)
    grid = (N_NODES + BLK - 1) // BLK
    pk_spec = pl.BlockSpec((PBLK, D_FEAT), lambda i: (i, 0))
    full = lambda shape: pl.BlockSpec(shape, lambda i: tuple(0 for _ in shape))
    return pl.pallas_call(
        _mlp_kernel,
        grid=(grid,),
        in_specs=[
            pl.BlockSpec((BLK, D_FEAT), lambda i: (i, 0)),
            pk_spec, pk_spec, pk_spec, pk_spec,
            full((D_FEAT, D_FEAT)),
            full((D_FEAT, PACK * D_FEAT)),
            full((D_FEAT, PACK * D_FEAT)),
            full((1, D_FEAT)),
            full((D_FEAT, D_FEAT)),
            full((1, D_FEAT)),
        ],
        out_specs=pl.BlockSpec((BLK, D_FEAT), lambda i: (i, 0)),
        out_shape=jax.ShapeDtypeStruct((N_NODES, D_FEAT), jnp.float32),
    )(nodes, s0, s1, r0, r1, w1n, wbs, wbr, b1r, W2, b2r)


def kernel(nodes, edges, senders, receivers, W1, b1, W2, b2):
    edges_nat = (edges.T.reshape(2, 8, GROUPS, 128)
                 .transpose(0, 2, 1, 3).reshape(-1))
    s0, r0, s1, r1 = _sc_segment_sums(edges_nat, senders.astype(jnp.int32),
                                      receivers.astype(jnp.int32))
    pk = ACC_ROWS * D_EDGE // D_FEAT
    return _tc_mlp(nodes, s0.reshape(pk, D_FEAT), r0.reshape(pk, D_FEAT),
                   s1.reshape(pk, D_FEAT), r1.reshape(pk, D_FEAT),
                   W1, b1, W2, b2)


# speedup vs baseline: 22.4160x; 1.1044x over previous
import functools

import jax
import jax.numpy as jnp
from jax import lax
from jax.experimental import pallas as pl
from jax.experimental.pallas import tpu as pltpu
from jax.experimental.pallas import tpu_sc as plsc

N_NODES = 10000
N_EDGES = 320000
D_FEAT = 128
D_EDGE = 16

NC = 2
NS = 16
NW = NC * NS

IDXW = 128
GROUPS = N_EDGES // IDXW
GROUPS_PER_TILE = GROUPS // NW
GROUPS_MAIN = GROUPS_PER_TILE * NW
GROUPS_REM = GROUPS - GROUPS_MAIN
CHUNK = 6
NCHUNKS = GROUPS_PER_TILE // CHUNK
XROWS = CHUNK * 8
HALF_OFF = N_EDGES // D_EDGE

ACC_ROWS = 10240
STRIPE = ACC_ROWS // NS


def _sc_segment_sums(edges_nat, senders, receivers):
    mesh = plsc.VectorSubcoreMesh(core_axis_name="c", subcore_axis_name="s")

    out_t = jax.ShapeDtypeStruct((ACC_ROWS, D_EDGE), jnp.float32)
    XLEN = 2 * CHUNK * 8 * 128
    ILEN = CHUNK * IDXW
    SCAT_BYTES = IDXW * D_EDGE * 4

    @functools.partial(
        pl.kernel,
        out_type=[out_t, out_t, out_t, out_t],
        mesh=mesh,
        compiler_params=pltpu.CompilerParams(use_tc_tiling_on_sc=False,
                                             needs_layout_passes=False),
        scratch_types=[
            pltpu.VMEM((XLEN,), jnp.float32),
            pltpu.VMEM((XLEN,), jnp.float32),
            pltpu.VMEM((ILEN, D_EDGE), jnp.float32),
            pltpu.VMEM((ILEN, D_EDGE), jnp.float32),
            pltpu.VMEM((ILEN,), jnp.int32),
            pltpu.VMEM((ILEN,), jnp.int32),
            pltpu.VMEM((ILEN,), jnp.int32),
            pltpu.VMEM((ILEN,), jnp.int32),
            pltpu.VMEM((ILEN,), jnp.int32),
            pltpu.VMEM((ILEN,), jnp.int32),
            pltpu.VMEM((STRIPE, D_EDGE), jnp.float32),
            pltpu.VMEM_SHARED((ACC_ROWS, D_EDGE), jnp.float32),
            pltpu.VMEM_SHARED((ACC_ROWS, D_EDGE), jnp.float32),
            pltpu.SemaphoreType.DMA,
            pltpu.SemaphoreType.DMA,
            pltpu.SemaphoreType.DMA,
        ],
    )
    def seg_kernel(edges_hbm, s_hbm, r_hbm, out_s0, out_r0, out_s1, out_r1,
                   xbuf0, xbuf1, ebuf0, ebuf1, sb0, sb1, sb2, rb0, rb1, rb2,
                   tbuf, sent_acc, recv_acc, sem_in, sem_sc0, sem_sc1):
        cid = lax.axis_index("c")
        sid = lax.axis_index("s")
        wid = cid * NS + sid

        xbuf = [xbuf0, xbuf1]
        ebuf = [ebuf0, ebuf1]
        sbuf = [sb0, sb1, sb2]
        rbuf = [rb0, rb1, rb2]
        sem_sc = [sem_sc0, sem_sc1]

        @pl.loop(0, STRIPE)
        def _(i):
            tbuf[i, :] = jnp.zeros((D_EDGE,), jnp.float32)

        base = sid * STRIPE
        pltpu.sync_copy(tbuf, sent_acc.at[pl.ds(base, STRIPE)])
        pltpu.sync_copy(tbuf, recv_acc.at[pl.ds(base, STRIPE)])
        plsc.subcore_barrier()

        iota = lax.iota(jnp.int32, 16)

        def issue_inputs(ci):
            g0 = wid * GROUPS_PER_TILE + ci * CHUNK
            e0 = g0 * IDXW
            xb = xbuf[ci % 2]
            return [
                pltpu.async_copy(edges_hbm.at[pl.ds(1024 * g0, 1024 * CHUNK)],
                                 xb.at[pl.ds(0, 1024 * CHUNK)], sem_in),
                pltpu.async_copy(
                    edges_hbm.at[pl.ds(128 * HALF_OFF + 1024 * g0,
                                       1024 * CHUNK)],
                    xb.at[pl.ds(CHUNK * 1024, 1024 * CHUNK)], sem_in),
                pltpu.async_copy(s_hbm.at[pl.ds(e0, ILEN)], sbuf[ci % 3],
                                 sem_in),
                pltpu.async_copy(r_hbm.at[pl.ds(e0, ILEN)], rbuf[ci % 3],
                                 sem_in),
            ]

        def drain_scatters(parity):
            for _ in range(2):
                pltpu.make_async_copy(edges_hbm.at[pl.ds(0, XLEN)],
                                      xbuf[0], sem_sc[parity]).wait()
        assert 2 * CHUNK * SCAT_BYTES == 2 * XLEN * 4

        in_handles = issue_inputs(0)
        for ci in range(NCHUNKS):
            b = ci % 2
            for h in in_handles:
                h.wait()
            if ci >= 2:
                drain_scatters(b)
            if ci + 1 < NCHUNKS:
                in_handles = issue_inputs(ci + 1)
            xb, eb = xbuf[b], ebuf[b]
            sb, rb = sbuf[ci % 3], rbuf[ci % 3]

            @plsc.parallel_loop(0, CHUNK * (IDXW // 16))
            def _(k):
                j = k >> 3
                c0 = k & 7
                for d in range(D_EDGE):
                    xoff = (d >> 3) * (CHUNK * 1024) + (d & 7) * 128
                    cvec = jnp.full((16,), d, jnp.int32)
                    v = xb[pl.ds(1024 * j + xoff + 16 * c0, 16)]
                    dst = eb.at[pl.ds(IDXW * j + 16 * c0, 16)]
                    plsc.store_scatter(dst, [iota, cvec], v)

            @pl.loop(0, CHUNK)
            def _(j):
                e = eb.at[pl.ds(j * IDXW, IDXW)]
                pltpu.async_copy(e, sent_acc.at[sb.at[pl.ds(j * IDXW, IDXW)]],
                                 sem_sc[b], add=True)
                pltpu.async_copy(e, recv_acc.at[rb.at[pl.ds(j * IDXW, IDXW)]],
                                 sem_sc[b], add=True)

        drain_scatters(NCHUNKS % 2)
        drain_scatters((NCHUNKS - 1) % 2)

        @pl.when(wid < GROUPS_REM)
        def _():
            g0 = GROUPS_MAIN + wid
            e0 = g0 * IDXW
            pltpu.sync_copy(edges_hbm.at[pl.ds(1024 * g0, 1024)],
                            xbuf0.at[pl.ds(0, 1024)])
            pltpu.sync_copy(
                edges_hbm.at[pl.ds(128 * HALF_OFF + 1024 * g0, 1024)],
                xbuf0.at[pl.ds(CHUNK * 1024, 1024)])
            pltpu.sync_copy(s_hbm.at[pl.ds(e0, IDXW)],
                            sb0.at[pl.ds(0, IDXW)])
            pltpu.sync_copy(r_hbm.at[pl.ds(e0, IDXW)],
                            rb0.at[pl.ds(0, IDXW)])
            for d in range(D_EDGE):
                xoff = (d >> 3) * (CHUNK * 1024) + (d & 7) * 128
                cvec = jnp.full((16,), d, jnp.int32)
                for c0 in range(IDXW // 16):
                    v = xbuf0[pl.ds(xoff + 16 * c0, 16)]
                    plsc.store_scatter(ebuf0.at[pl.ds(16 * c0, 16)],
                                       [iota, cvec], v)
            e = ebuf0.at[pl.ds(0, IDXW)]
            pltpu.sync_copy(e, sent_acc.at[sb0.at[pl.ds(0, IDXW)]], add=True)
            pltpu.sync_copy(e, recv_acc.at[rb0.at[pl.ds(0, IDXW)]], add=True)

        plsc.subcore_barrier()

        def copy_out(acc, out_ref):
            pltpu.sync_copy(acc.at[pl.ds(base, STRIPE)], tbuf)
            pltpu.sync_copy(tbuf, out_ref.at[pl.ds(base, STRIPE)])

        @pl.when(cid == 0)
        def _():
            copy_out(sent_acc, out_s0)
            copy_out(recv_acc, out_r0)

        @pl.when(cid == 1)
        def _():
            copy_out(sent_acc, out_s1)
            copy_out(recv_acc, out_r1)

    return seg_kernel(edges_nat, senders, receivers)


BLK = 1024
PACK = D_FEAT // D_EDGE
PBLK = BLK // PACK


def _mlp_kernel(nodes_ref, s0_ref, s1_ref, r0_ref, r1_ref,
                w1n_ref, wbs_ref, wbr_ref, b1_ref, w2_ref, b2_ref, out_ref):
    sent = s0_ref[...] + s1_ref[...]
    recv = r0_ref[...] + r1_ref[...]
    hs = jnp.dot(sent, wbs_ref[...],
                 preferred_element_type=jnp.float32).reshape(BLK, D_FEAT)
    hr = jnp.dot(recv, wbr_ref[...],
                 preferred_element_type=jnp.float32).reshape(BLK, D_FEAT)
    h = (jnp.dot(nodes_ref[...], w1n_ref[...],
                 preferred_element_type=jnp.float32) + hs + hr + b1_ref[...])
    h = jnp.maximum(h, 0.0)
    out_ref[...] = (jnp.dot(h, w2_ref[...], preferred_element_type=jnp.float32)
                    + b2_ref[...])


def _tc_mlp(nodes, s0, r0, s1, r1, W1, b1, W2, b2):
    w1n = W1[:D_FEAT]
    w1s = W1[D_FEAT:D_FEAT + D_EDGE]
    w1r = W1[D_FEAT + D_EDGE:]
    eye8 = jnp.eye(PACK, dtype=jnp.float32)
    wbs = jnp.einsum("ab,dj->adbj", eye8, w1s).reshape(D_FEAT, PACK * D_FEAT)
    wbr = jnp.einsum("ab,dj->adbj", eye8, w1r).reshape(D_FEAT, PACK * D_FEAT)
    b1r = b1.reshape(1, -1)
    b2r = b2.reshape(1, -1)
    grid = (N_NODES + BLK - 1) // BLK
    pk_spec = pl.BlockSpec((PBLK, D_FEAT), lambda i: (i, 0))
    full = lambda shape: pl.BlockSpec(shape, lambda i: tuple(0 for _ in shape))
    return pl.pallas_call(
        _mlp_kernel,
        grid=(grid,),
        in_specs=[
            pl.BlockSpec((BLK, D_FEAT), lambda i: (i, 0)),
            pk_spec, pk_spec, pk_spec, pk_spec,
            full((D_FEAT, D_FEAT)),
            full((D_FEAT, PACK * D_FEAT)),
            full((D_FEAT, PACK * D_FEAT)),
            full((1, D_FEAT)),
            full((D_FEAT, D_FEAT)),
            full((1, D_FEAT)),
        ],
        out_specs=pl.BlockSpec((BLK, D_FEAT), lambda i: (i, 0)),
        out_shape=jax.ShapeDtypeStruct((N_NODES, D_FEAT), jnp.float32),
    )(nodes, s0, s1, r0, r1, w1n, wbs, wbr, b1r, W2, b2r)


def kernel(nodes, edges, senders, receivers, W1, b1, W2, b2):
    edges_nat = (edges.T.reshape(2, 8, GROUPS, 128)
                 .transpose(0, 2, 1, 3).reshape(-1))
    s0, r0, s1, r1 = _sc_segment_sums(edges_nat, senders.astype(jnp.int32),
                                      receivers.astype(jnp.int32))
    pk = ACC_ROWS * D_EDGE // D_FEAT
    return _tc_mlp(nodes, s0.reshape(pk, D_FEAT), r0.reshape(pk, D_FEAT),
                   s1.reshape(pk, D_FEAT), r1.reshape(pk, D_FEAT),
                   W1, b1, W2, b2)
